# baseline (speedup 1.0000x reference)
import functools

import jax
import jax.numpy as jnp
from jax.experimental import pallas as pl
from jax.experimental.pallas import tpu as pltpu

POOL = 2
VMEM_LIMIT = 48 * 1024 * 1024
G_CONV1 = 2
G_CONV2 = 2


def _sig(x):
    return pl.reciprocal(1.0 + jnp.exp(-x), approx=False)


def _conv_body(G, hp, kt, N, x_ref, t_ref, b_ref, o_ref):
    acc = None
    for j in range(kt):
        rows = [x_ref[g, j: j + hp] for g in range(G)]
        band = rows[0] if G == 1 else jnp.concatenate(rows, axis=0)
        d = jnp.dot(band, t_ref[j], preferred_element_type=jnp.float32)
        acc = d if acc is None else acc + d
    n2 = 2 * N
    m = jnp.maximum(acc[:, :n2], acc[:, n2:])
    m = jnp.maximum(m[:, :N], m[:, N:])
    o_ref[...] = _sig(m + b_ref[...]).astype(o_ref.dtype).reshape(G, hp, N)


def _conv_stage(x_srows, t_sup, b_row, G):
    B, Hs, WC2 = x_srows.shape
    kt, _, N4 = t_sup.shape
    N = N4 // 4
    hp = Hs - kt + 1
    return pl.pallas_call(
        functools.partial(_conv_body, G, hp, kt, N),
        out_shape=jax.ShapeDtypeStruct((B, hp, N), jnp.bfloat16),
        grid=(B // G,),
        in_specs=[
            pl.BlockSpec((G, Hs, WC2), lambda i: (i, 0, 0)),
            pl.BlockSpec((kt, WC2, N4), lambda i: (0, 0, 0)),
            pl.BlockSpec((1, N), lambda i: (0, 0)),
        ],
        out_specs=pl.BlockSpec((G, hp, N), lambda i: (i, 0, 0)),
        compiler_params=pltpu.CompilerParams(
            dimension_semantics=("parallel",),
            vmem_limit_bytes=VMEM_LIMIT),
    )(x_srows, t_sup, b_row)


def _fc_body(x_ref, w1_ref, b1_ref, w2_ref, b2_ref, w3_ref, b3_ref, o_ref):
    h1 = _sig(jnp.dot(x_ref[...], w1_ref[...],
                      preferred_element_type=jnp.float32) + b1_ref[...])
    h2 = _sig(jnp.dot(h1, w2_ref[...],
                      preferred_element_type=jnp.float32) + b2_ref[...])
    o_ref[...] = (jnp.dot(h2, w3_ref[...],
                          preferred_element_type=jnp.float32) + b3_ref[...])


def _fc_stage(feat, w1, b1, w2, b2, w3, b3):
    MB, K = feat.shape
    H1, H2, NC = w1.shape[1], w2.shape[1], w3.shape[1]
    MT = MB // 2 if MB % 16 == 0 else MB
    return pl.pallas_call(
        _fc_body,
        out_shape=jax.ShapeDtypeStruct((MB, NC), jnp.float32),
        grid=(MB // MT,),
        in_specs=[
            pl.BlockSpec((MT, K), lambda i: (i, 0)),
            pl.BlockSpec((K, H1), lambda i: (0, 0)),
            pl.BlockSpec((1, H1), lambda i: (0, 0)),
            pl.BlockSpec((H1, H2), lambda i: (0, 0)),
            pl.BlockSpec((1, H2), lambda i: (0, 0)),
            pl.BlockSpec((H2, NC), lambda i: (0, 0)),
            pl.BlockSpec((1, NC), lambda i: (0, 0)),
        ],
        out_specs=pl.BlockSpec((MT, NC), lambda i: (i, 0)),
        compiler_params=pltpu.CompilerParams(
            dimension_semantics=("parallel",),
            vmem_limit_bytes=VMEM_LIMIT),
    )(feat, w1, b1.reshape(1, H1), w2, b2.reshape(1, H2), w3, b3.reshape(1, NC))


def _prep_weights(t, perm_cw=None):
    kh, WC, N = t.shape[1], t.shape[2], t.shape[3]
    if perm_cw is not None:
        C, W = perm_cw
        t = t.reshape(2, kh, W, C, N)
        t = jnp.transpose(t, (0, 1, 3, 2, 4)).reshape(2, kh, WC, N)
    z = jnp.zeros((2, 1, WC, N), t.dtype)
    s0 = jnp.concatenate([t, z], axis=1)
    s1 = jnp.concatenate([z, t], axis=1)
    s = jnp.stack([s0, s1], axis=0)
    s = jnp.transpose(s, (2, 3, 0, 1, 4))
    kt = (kh + 1) // 2
    return s.reshape(kt, 2 * WC, 4 * N).astype(jnp.bfloat16)


def kernel(x, t1, b1, t2, b2, fc1_w, fc1_b, fc2_w, fc2_b, fc3_w, fc3_b):
    B, C, H, W = x.shape
    xr = jnp.transpose(x, (0, 2, 1, 3)).reshape(B, H // 2, 2 * C * W)
    xr = xr.astype(jnp.bfloat16)

    y1 = _conv_stage(xr, _prep_weights(t1, (C, W)), b1,
                     G_CONV1)
    y1s = y1.reshape(B, y1.shape[1] // 2, 2 * y1.shape[2])
    y2 = _conv_stage(y1s, _prep_weights(t2), b2, G_CONV2)

    feat = y2.reshape(B, y2.shape[1] * y2.shape[2])
    return _fc_stage(feat, fc1_w.astype(jnp.bfloat16), fc1_b,
                     fc2_w, fc2_b, fc3_w, fc3_b)


# speedup vs baseline: 1.1082x; 1.1082x over previous
import functools

import jax
import jax.numpy as jnp
from jax.experimental import pallas as pl
from jax.experimental.pallas import tpu as pltpu

POOL = 2
VMEM_LIMIT = 48 * 1024 * 1024
G_CONV1 = 2
G_CONV2 = 4


def _sig(x):
    return pl.reciprocal(1.0 + jnp.exp(-x), approx=False)


def _conv_body(G, hp, kt, N, split_c, x_ref, t_ref, b_ref, o_ref):
    if split_c:
        C = x_ref.shape[1]
        sup = [jnp.concatenate(
            [x_ref[g, c].astype(jnp.bfloat16) for c in range(C)], axis=1)
            for g in range(G)]
    else:
        sup = [x_ref[g] for g in range(G)]
    acc = None
    for j in range(kt):
        rows = [sup[g][j: j + hp] for g in range(G)]
        band = rows[0] if G == 1 else jnp.concatenate(rows, axis=0)
        d = jnp.dot(band, t_ref[j], preferred_element_type=jnp.float32)
        acc = d if acc is None else acc + d
    n2 = 2 * N
    m = jnp.maximum(acc[:, :n2], acc[:, n2:])
    m = jnp.maximum(m[:, :N], m[:, N:])
    o_ref[...] = _sig(m + b_ref[...]).astype(o_ref.dtype).reshape(G, hp, N)


def _conv_stage(x_in, t_sup, b_row, G):
    split_c = x_in.ndim == 4
    B = x_in.shape[0]
    kt, WC2, N4 = t_sup.shape
    N = N4 // 4
    Hs = x_in.shape[2] if split_c else x_in.shape[1]
    hp = Hs - kt + 1
    if split_c:
        in_spec = pl.BlockSpec((G,) + x_in.shape[1:], lambda i: (i, 0, 0, 0))
    else:
        in_spec = pl.BlockSpec((G, Hs, WC2), lambda i: (i, 0, 0))
    return pl.pallas_call(
        functools.partial(_conv_body, G, hp, kt, N, split_c),
        out_shape=jax.ShapeDtypeStruct((B, hp, N), jnp.bfloat16),
        grid=(B // G,),
        in_specs=[
            in_spec,
            pl.BlockSpec((kt, WC2, N4), lambda i: (0, 0, 0)),
            pl.BlockSpec((1, N), lambda i: (0, 0)),
        ],
        out_specs=pl.BlockSpec((G, hp, N), lambda i: (i, 0, 0)),
        compiler_params=pltpu.CompilerParams(
            dimension_semantics=("parallel",),
            vmem_limit_bytes=VMEM_LIMIT),
    )(x_in, t_sup, b_row)


def _fc_body(x_ref, w1_ref, b1_ref, w2_ref, b2_ref, w3_ref, b3_ref, o_ref):
    w1 = w1_ref[...].astype(jnp.bfloat16)
    h1 = _sig(jnp.dot(x_ref[...], w1,
                      preferred_element_type=jnp.float32) + b1_ref[...])
    h2 = _sig(jnp.dot(h1, w2_ref[...],
                      preferred_element_type=jnp.float32) + b2_ref[...])
    o_ref[...] = (jnp.dot(h2, w3_ref[...],
                          preferred_element_type=jnp.float32) + b3_ref[...])


def _fc_stage(feat, w1, b1, w2, b2, w3, b3):
    MB, K = feat.shape
    H1, H2, NC = w1.shape[1], w2.shape[1], w3.shape[1]
    MT = MB // 2 if MB % 16 == 0 else MB
    return pl.pallas_call(
        _fc_body,
        out_shape=jax.ShapeDtypeStruct((MB, NC), jnp.float32),
        grid=(MB // MT,),
        in_specs=[
            pl.BlockSpec((MT, K), lambda i: (i, 0)),
            pl.BlockSpec((K, H1), lambda i: (0, 0)),
            pl.BlockSpec((1, H1), lambda i: (0, 0)),
            pl.BlockSpec((H1, H2), lambda i: (0, 0)),
            pl.BlockSpec((1, H2), lambda i: (0, 0)),
            pl.BlockSpec((H2, NC), lambda i: (0, 0)),
            pl.BlockSpec((1, NC), lambda i: (0, 0)),
        ],
        out_specs=pl.BlockSpec((MT, NC), lambda i: (i, 0)),
        compiler_params=pltpu.CompilerParams(
            dimension_semantics=("parallel",),
            vmem_limit_bytes=VMEM_LIMIT),
    )(feat, w1, b1.reshape(1, H1), w2, b2.reshape(1, H2), w3, b3.reshape(1, NC))


def _prep_weights(t, perm_wc=None):
    kh, WC, N = t.shape[1], t.shape[2], t.shape[3]
    kt = (kh + 1) // 2
    z = jnp.zeros((2, 1, WC, N), t.dtype)
    s0 = jnp.concatenate([t, z], axis=1)
    s1 = jnp.concatenate([z, t], axis=1)
    a = jnp.stack([s0, s1], axis=0)
    if perm_wc is None:
        a = a.reshape(2, 2, kt, 2, WC, N)
        a = jnp.transpose(a, (2, 3, 4, 0, 1, 5))
    else:
        W, C = perm_wc
        a = a.reshape(2, 2, kt, 2, W, C, N)
        a = jnp.transpose(a, (2, 5, 3, 4, 0, 1, 6))
    return a.reshape(kt, 2 * WC, 4 * N).astype(jnp.bfloat16)


def kernel(x, t1, b1, t2, b2, fc1_w, fc1_b, fc2_w, fc2_b, fc3_w, fc3_b):
    B, C, H, W = x.shape
    xs = x.reshape(B, C, H // 2, 2 * W)

    y1 = _conv_stage(xs, _prep_weights(t1, (W, C)), b1,
                     G_CONV1)
    y1s = y1.reshape(B, y1.shape[1] // 2, 2 * y1.shape[2])
    y2 = _conv_stage(y1s, _prep_weights(t2), b2, G_CONV2)

    feat = y2.reshape(B, y2.shape[1] * y2.shape[2])
    return _fc_stage(feat, fc1_w, fc1_b, fc2_w, fc2_b, fc3_w, fc3_b)


# speedup vs baseline: 1.3148x; 1.1864x over previous
import functools

import jax
import jax.numpy as jnp
from jax.experimental import pallas as pl
from jax.experimental.pallas import tpu as pltpu

POOL = 2
VMEM_LIMIT = 48 * 1024 * 1024
G_CONV1 = 2
G_CONV2 = 4


def _sig(x):
    return pl.reciprocal(1.0 + jnp.exp(-x), approx=False)


def _conv_body(G, hp, kt, N, split_c, x_ref, t_ref, b_ref, o_ref):
    if split_c:
        C = x_ref.shape[1]
        sup = [jnp.concatenate(
            [x_ref[g, c].astype(jnp.bfloat16) for c in range(C)], axis=1)
            for g in range(G)]
    else:
        sup = [x_ref[g] for g in range(G)]
    acc = None
    for j in range(kt):
        rows = [sup[g][j: j + hp] for g in range(G)]
        band = rows[0] if G == 1 else jnp.concatenate(rows, axis=0)
        d = jnp.dot(band, t_ref[j], preferred_element_type=jnp.float32)
        acc = d if acc is None else acc + d
    n2 = 2 * N
    m = jnp.maximum(acc[:, :n2], acc[:, n2:])
    m = jnp.maximum(m[:, :N], m[:, N:])
    o_ref[...] = _sig(m + b_ref[...]).astype(o_ref.dtype).reshape(G, hp, N)


def _conv_stage(x_in, t_sup, b_row, G):
    split_c = x_in.ndim == 4
    B = x_in.shape[0]
    kt, WC2, N4 = t_sup.shape
    N = N4 // 4
    Hs = x_in.shape[2] if split_c else x_in.shape[1]
    hp = Hs - kt + 1
    if split_c:
        in_spec = pl.BlockSpec((G,) + x_in.shape[1:], lambda i: (i, 0, 0, 0))
    else:
        in_spec = pl.BlockSpec((G, Hs, WC2), lambda i: (i, 0, 0))
    return pl.pallas_call(
        functools.partial(_conv_body, G, hp, kt, N, split_c),
        out_shape=jax.ShapeDtypeStruct((B, hp, N), jnp.bfloat16),
        grid=(B // G,),
        in_specs=[
            in_spec,
            pl.BlockSpec((kt, WC2, N4), lambda i: (0, 0, 0)),
            pl.BlockSpec((1, N), lambda i: (0, 0)),
        ],
        out_specs=pl.BlockSpec((G, hp, N), lambda i: (i, 0, 0)),
        compiler_params=pltpu.CompilerParams(
            dimension_semantics=("parallel",),
            vmem_limit_bytes=VMEM_LIMIT),
    )(x_in, t_sup, b_row)


def _fc_body(x_ref, w1_ref, b1_ref, w2_ref, b2_ref, w3_ref, b3_ref, o_ref):
    w1 = w1_ref[...].astype(jnp.bfloat16)
    h1 = _sig(jnp.dot(x_ref[...], w1,
                      preferred_element_type=jnp.float32) + b1_ref[...])
    h2 = _sig(jnp.dot(h1, w2_ref[...],
                      preferred_element_type=jnp.float32) + b2_ref[...])
    o_ref[...] = (jnp.dot(h2, w3_ref[...],
                          preferred_element_type=jnp.float32) + b3_ref[...])


def _fc_stage(feat, w1, b1, w2, b2, w3, b3):
    MB, K = feat.shape
    H1, H2, NC = w1.shape[1], w2.shape[1], w3.shape[1]
    MT = MB // 2 if MB % 16 == 0 else MB
    return pl.pallas_call(
        _fc_body,
        out_shape=jax.ShapeDtypeStruct((MB, NC), jnp.float32),
        grid=(MB // MT,),
        in_specs=[
            pl.BlockSpec((MT, K), lambda i: (i, 0)),
            pl.BlockSpec((K, H1), lambda i: (0, 0)),
            pl.BlockSpec((1, H1), lambda i: (0, 0)),
            pl.BlockSpec((H1, H2), lambda i: (0, 0)),
            pl.BlockSpec((1, H2), lambda i: (0, 0)),
            pl.BlockSpec((H2, NC), lambda i: (0, 0)),
            pl.BlockSpec((1, NC), lambda i: (0, 0)),
        ],
        out_specs=pl.BlockSpec((MT, NC), lambda i: (i, 0)),
        compiler_params=pltpu.CompilerParams(
            dimension_semantics=("parallel",),
            vmem_limit_bytes=VMEM_LIMIT),
    )(feat, w1, b1.reshape(1, H1), w2, b2.reshape(1, H2), w3, b3.reshape(1, NC))


def _prep2_body(kh, WC, N, t_ref, o_ref):
    kt = (kh + 1) // 2
    zero = jnp.zeros((WC, N), jnp.bfloat16)
    for j in range(kt):
        for s in range(2):
            blocks = []
            for dh in range(2):
                i = 2 * j + s - dh
                for dw in range(2):
                    blocks.append(t_ref[dw, i].astype(jnp.bfloat16)
                                  if 0 <= i < kh else zero)
            o_ref[j, s * WC:(s + 1) * WC, :] = jnp.concatenate(blocks, axis=1)


def _prep2(t):
    kh, WC, N = t.shape[1], t.shape[2], t.shape[3]
    kt = (kh + 1) // 2
    return pl.pallas_call(
        functools.partial(_prep2_body, kh, WC, N),
        out_shape=jax.ShapeDtypeStruct((kt, 2 * WC, 4 * N), jnp.bfloat16),
        compiler_params=pltpu.CompilerParams(
            vmem_limit_bytes=VMEM_LIMIT),
    )(t)


def _prep_weights(t, perm_wc=None):
    kh, WC, N = t.shape[1], t.shape[2], t.shape[3]
    kt = (kh + 1) // 2
    z = jnp.zeros((2, 1, WC, N), t.dtype)
    s0 = jnp.concatenate([t, z], axis=1)
    s1 = jnp.concatenate([z, t], axis=1)
    a = jnp.stack([s0, s1], axis=0)
    if perm_wc is None:
        a = a.reshape(2, 2, kt, 2, WC, N)
        a = jnp.transpose(a, (2, 3, 4, 0, 1, 5))
    else:
        W, C = perm_wc
        a = a.reshape(2, 2, kt, 2, W, C, N)
        a = jnp.transpose(a, (2, 5, 3, 4, 0, 1, 6))
    return a.reshape(kt, 2 * WC, 4 * N).astype(jnp.bfloat16)


def kernel(x, t1, b1, t2, b2, fc1_w, fc1_b, fc2_w, fc2_b, fc3_w, fc3_b):
    B, C, H, W = x.shape
    xs = x.reshape(B, C, H // 2, 2 * W)

    y1 = _conv_stage(xs, _prep_weights(t1, (W, C)), b1,
                     G_CONV1)
    y1s = y1.reshape(B, y1.shape[1] // 2, 2 * y1.shape[2])
    y2 = _conv_stage(y1s, _prep2(t2), b2, G_CONV2)

    feat = y2.reshape(B, y2.shape[1] * y2.shape[2])
    return _fc_stage(feat, fc1_w, fc1_b, fc2_w, fc2_b, fc3_w, fc3_b)


# speedup vs baseline: 1.6654x; 1.2666x over previous
import functools

import jax
import jax.numpy as jnp
from jax.experimental import pallas as pl
from jax.experimental.pallas import tpu as pltpu

POOL = 2
VMEM_LIMIT = 48 * 1024 * 1024
G_CONV1 = 2
G_CONV2 = 4


def _sig(x):
    return pl.reciprocal(1.0 + jnp.exp(-x), approx=False)


def _conv_body(G, hp, kt, N, split_c, x_ref, t_ref, b_ref, o_ref):
    if split_c:
        C = x_ref.shape[1]
        sup = [jnp.concatenate(
            [x_ref[g, c].astype(jnp.bfloat16) for c in range(C)], axis=1)
            for g in range(G)]
    else:
        sup = [x_ref[g] for g in range(G)]
    acc = None
    for j in range(kt):
        rows = [sup[g][j: j + hp] for g in range(G)]
        band = rows[0] if G == 1 else jnp.concatenate(rows, axis=0)
        d = jnp.dot(band, t_ref[j], preferred_element_type=jnp.float32)
        acc = d if acc is None else acc + d
    n2 = 2 * N
    m = jnp.maximum(acc[:, :n2], acc[:, n2:])
    m = jnp.maximum(m[:, :N], m[:, N:])
    o_ref[...] = _sig(m + b_ref[...]).astype(o_ref.dtype).reshape(G, hp, N)


def _conv_stage(x_in, t_sup, b_row, G):
    split_c = x_in.ndim == 4
    B = x_in.shape[0]
    kt, WC2, N4 = t_sup.shape
    N = N4 // 4
    Hs = x_in.shape[2] if split_c else x_in.shape[1]
    hp = Hs - kt + 1
    if split_c:
        in_spec = pl.BlockSpec((G,) + x_in.shape[1:], lambda i: (i, 0, 0, 0))
    else:
        in_spec = pl.BlockSpec((G, Hs, WC2), lambda i: (i, 0, 0))
    return pl.pallas_call(
        functools.partial(_conv_body, G, hp, kt, N, split_c),
        out_shape=jax.ShapeDtypeStruct((B, hp, N), jnp.bfloat16),
        grid=(B // G,),
        in_specs=[
            in_spec,
            pl.BlockSpec((kt, WC2, N4), lambda i: (0, 0, 0)),
            pl.BlockSpec((1, N), lambda i: (0, 0)),
        ],
        out_specs=pl.BlockSpec((G, hp, N), lambda i: (i, 0, 0)),
        compiler_params=pltpu.CompilerParams(
            dimension_semantics=("parallel",),
            vmem_limit_bytes=VMEM_LIMIT),
    )(x_in, t_sup, b_row)


def _fc_body(x_ref, w1_ref, b1_ref, w2_ref, b2_ref, w3_ref, b3_ref, o_ref):
    w1 = w1_ref[...].astype(jnp.bfloat16)
    h1 = _sig(jnp.dot(x_ref[...], w1,
                      preferred_element_type=jnp.float32) + b1_ref[...])
    h2 = _sig(jnp.dot(h1, w2_ref[...],
                      preferred_element_type=jnp.float32) + b2_ref[...])
    o_ref[...] = (jnp.dot(h2, w3_ref[...],
                          preferred_element_type=jnp.float32) + b3_ref[...])


def _fc_stage(feat, w1, b1, w2, b2, w3, b3):
    MB, K = feat.shape
    H1, H2, NC = w1.shape[1], w2.shape[1], w3.shape[1]
    MT = MB // 2 if MB % 16 == 0 else MB
    return pl.pallas_call(
        _fc_body,
        out_shape=jax.ShapeDtypeStruct((MB, NC), jnp.float32),
        grid=(MB // MT,),
        in_specs=[
            pl.BlockSpec((MT, K), lambda i: (i, 0)),
            pl.BlockSpec((K, H1), lambda i: (0, 0)),
            pl.BlockSpec((1, H1), lambda i: (0, 0)),
            pl.BlockSpec((H1, H2), lambda i: (0, 0)),
            pl.BlockSpec((1, H2), lambda i: (0, 0)),
            pl.BlockSpec((H2, NC), lambda i: (0, 0)),
            pl.BlockSpec((1, NC), lambda i: (0, 0)),
        ],
        out_specs=pl.BlockSpec((MT, NC), lambda i: (i, 0)),
        compiler_params=pltpu.CompilerParams(
            dimension_semantics=("parallel",),
            vmem_limit_bytes=VMEM_LIMIT),
    )(feat, w1, b1.reshape(1, H1), w2, b2.reshape(1, H2), w3, b3.reshape(1, NC))


def _prep2_body(kh, WC, N, t_ref, o_ref):
    kt = (kh + 1) // 2
    zero = jnp.zeros((WC, N), jnp.bfloat16)
    for j in range(kt):
        for s in range(2):
            blocks = []
            for dh in range(2):
                i = 2 * j + s - dh
                for dw in range(2):
                    blocks.append(t_ref[dw, i].astype(jnp.bfloat16)
                                  if 0 <= i < kh else zero)
            o_ref[j, s * WC:(s + 1) * WC, :] = jnp.concatenate(blocks, axis=1)


def _prep2(t):
    kh, WC, N = t.shape[1], t.shape[2], t.shape[3]
    kt = (kh + 1) // 2
    return pl.pallas_call(
        functools.partial(_prep2_body, kh, WC, N),
        out_shape=jax.ShapeDtypeStruct((kt, 2 * WC, 4 * N), jnp.bfloat16),
        compiler_params=pltpu.CompilerParams(
            vmem_limit_bytes=VMEM_LIMIT),
    )(t)


def _prep1_body(kh, W, C, N, t_ref, o_ref):
    kt = (kh + 1) // 2
    WC = W * C
    r_out = jax.lax.broadcasted_iota(jnp.int32, (WC, WC), 0)
    r_in = jax.lax.broadcasted_iota(jnp.int32, (WC, WC), 1)
    perm = ((r_out % W) * C + r_out // W == r_in).astype(jnp.bfloat16)
    pb = [[jnp.dot(perm, t_ref[dw, i].astype(jnp.bfloat16),
                   preferred_element_type=jnp.float32).astype(jnp.bfloat16)
           for i in range(kh)] for dw in range(2)]
    zero = jnp.zeros((W, N), jnp.bfloat16)
    for j in range(kt):
        for c in range(C):
            rows_sw = []
            for s in range(2):
                lane_blocks = []
                for dh in range(2):
                    i = 2 * j + s - dh
                    for dw in range(2):
                        lane_blocks.append(pb[dw][i][c * W:(c + 1) * W]
                                           if 0 <= i < kh else zero)
                rows_sw.append(jnp.concatenate(lane_blocks, axis=1))
            o_ref[j, c * 2 * W:(c + 1) * 2 * W, :] = jnp.concatenate(
                rows_sw, axis=0)


def _prep1(t, W, C):
    kh, WC, N = t.shape[1], t.shape[2], t.shape[3]
    kt = (kh + 1) // 2
    return pl.pallas_call(
        functools.partial(_prep1_body, kh, W, C, N),
        out_shape=jax.ShapeDtypeStruct((kt, 2 * WC, 4 * N), jnp.bfloat16),
        compiler_params=pltpu.CompilerParams(
            vmem_limit_bytes=VMEM_LIMIT),
    )(t)


def kernel(x, t1, b1, t2, b2, fc1_w, fc1_b, fc2_w, fc2_b, fc3_w, fc3_b):
    B, C, H, W = x.shape
    xs = x.reshape(B, C, H // 2, 2 * W)

    y1 = _conv_stage(xs, _prep1(t1, W, C), b1,
                     G_CONV1)
    y1s = y1.reshape(B, y1.shape[1] // 2, 2 * y1.shape[2])
    y2 = _conv_stage(y1s, _prep2(t2), b2, G_CONV2)

    feat = y2.reshape(B, y2.shape[1] * y2.shape[2])
    return _fc_stage(feat, fc1_w, fc1_b, fc2_w, fc2_b, fc3_w, fc3_b)
